# baseline (speedup 1.0000x reference)
import functools

import jax
import jax.numpy as jnp
from jax import lax
from jax.experimental import pallas as pl
from jax.experimental.pallas import tpu as pltpu
from jax.experimental.pallas import tpu_sc as plsc

B, L, V, E, H, C = 16384, 5, 1000000, 64, 128, 2


_VBQ = 8192
_SQ = 253952
_TGRID = _SQ // _VBQ
_MAXBLK = (V // _VBQ) - 1


def _pack_body(x0_ref, x1_ref, x2_ref, x3_ref, eye_ref, o_ref):
    quarters = []
    for x_ref in (x0_ref, x1_ref, x2_ref, x3_ref):
        xb = lax.bitcast_convert_type(x_ref[...], jnp.int32)
        lo = jnp.bitwise_and(lax.shift_right_logical(xb[:E // 2, :], 16),
                             jnp.int32(0xFFFF))
        hi = jnp.bitwise_and(xb[E // 2:, :], jnp.int32(-65536))
        quarters.append(jnp.bitwise_or(lo, hi))
    p = jnp.concatenate(quarters, axis=0)
    pf = lax.bitcast_convert_type(p, jnp.float32)
    t = lax.dot_general(pf, eye_ref[...], (((0,), (0,)), ((), ())),
                        preferred_element_type=jnp.float32)
    o_ref[...] = lax.bitcast_convert_type(t, jnp.int32)


def _in_spec(m):
    return pl.BlockSpec(
        (E, _VBQ),
        lambda i, _m=m: (0, jnp.minimum(_m * _TGRID + i, _MAXBLK)))


_pack = pl.pallas_call(
    _pack_body,
    grid=(_TGRID,),
    in_specs=[_in_spec(m) for m in range(4)] + [
        pl.BlockSpec((2 * E, 2 * E), lambda i: (0, 0)),
    ],
    out_specs=pl.BlockSpec((_VBQ, 2 * E), lambda i: (i, 0)),
    out_shape=jax.ShapeDtypeStruct((_SQ, 2 * E), jnp.int32),
)


_NC, _NS = 2, 16
_NW = _NC * _NS
_N = B * L
_PW = _N // _NW
_CH = 128
_NCH = _PW // _CH
_NBUF = 6

_sc_mesh = plsc.VectorSubcoreMesh(core_axis_name="c", subcore_axis_name="s")


@functools.partial(
    pl.kernel,
    out_type=jax.ShapeDtypeStruct((_N, 2 * E), jnp.int32),
    mesh=_sc_mesh,
    scratch_types=[
        pltpu.VMEM((_NCH, _CH), jnp.int32),
        pltpu.VMEM((_NBUF, _CH, 2 * E), jnp.int32),
    ] + [pltpu.SemaphoreType.DMA] * _NBUF,
)
def _sc_gather(idx_hbm, tab_hbm, out_hbm, idx_v, rows_v, *sems):
    wid = lax.axis_index("s") * _NC + lax.axis_index("c")
    base = wid * _PW
    pltpu.sync_copy(idx_hbm.at[wid], idx_v)

    def start(i):
        bi = i % _NBUF
        return pltpu.async_copy(tab_hbm.at[idx_v.at[i]], rows_v.at[bi], sems[bi])

    handles = {}
    for i in range(min(_NBUF, _NCH)):
        handles[i] = start(i)
    for i in range(_NCH):
        bi = i % _NBUF
        handles.pop(i).wait()
        pltpu.sync_copy(rows_v.at[bi], out_hbm.at[pl.ds(base + i * _CH, _CH)])
        j = i + _NBUF
        if j < _NCH:
            handles[j] = start(j)


_BLK = 2048
_NB = B // _BLK


def _mlp_body(x_refs, q_refs, w1r_ref, b1_ref, w2_ref, b2_ref, w3_ref,
              b3_ref, o_ref):
    w1r = w1r_ref[...]
    acc = jnp.zeros((_BLK, H), jnp.float32)
    for l in range(L):
        xi = x_refs[l][...]
        qv = q_refs[l][...]
        laneq = lax.shift_right_logical(
            lax.broadcasted_iota(jnp.int32, (_BLK, 2 * E), 1), 5)
        masked = jnp.where(laneq == qv, xi, 0)
        f_lo = lax.bitcast_convert_type(
            lax.shift_left(masked, 16), jnp.float32).astype(jnp.bfloat16)
        f_hi = lax.bitcast_convert_type(
            jnp.bitwise_and(masked, jnp.int32(-65536)),
            jnp.float32).astype(jnp.bfloat16)
        acc = acc + jnp.dot(f_lo, w1r[2 * H * l:2 * H * l + H, :],
                            preferred_element_type=jnp.float32)
        acc = acc + jnp.dot(f_hi, w1r[2 * H * l + H:2 * H * (l + 1), :],
                            preferred_element_type=jnp.float32)
    h = jnp.maximum(acc + b1_ref[...], 0.0).astype(jnp.bfloat16)
    h = jnp.maximum(jnp.dot(h, w2_ref[...],
                            preferred_element_type=jnp.float32) + b2_ref[...], 0.0)
    o = jnp.dot(h.astype(jnp.bfloat16), w3_ref[...],
                preferred_element_type=jnp.float32) + b3_ref[...]
    col = lax.broadcasted_iota(jnp.int32, o.shape, 1)
    o = jnp.where(col < C, o, jnp.float32(-1e30))
    m = jnp.max(o, axis=1, keepdims=True)
    e = jnp.exp(o - m)
    o_ref[...] = e / jnp.sum(e, axis=1, keepdims=True)


def _mlp_entry(x0, x1, x2, x3, x4, q0, q1, q2, q3, q4,
               w1r, b1, w2, b2, w3, b3, o):
    _mlp_body((x0, x1, x2, x3, x4), (q0, q1, q2, q3, q4),
              w1r, b1, w2, b2, w3, b3, o)


def _x_spec(l):
    return pl.BlockSpec((_BLK, 2 * E), lambda i, _l=l: (_l * _NB + i, 0))


def _q_spec(l):
    return pl.BlockSpec((_BLK, 1), lambda i, _l=l: (_l * _NB + i, 0))


_mlp = pl.pallas_call(
    _mlp_entry,
    grid=(_NB,),
    in_specs=[_x_spec(l) for l in range(L)]
    + [_q_spec(l) for l in range(L)] + [
        pl.BlockSpec((2 * L * H, H), lambda i: (0, 0)),
        pl.BlockSpec((1, H), lambda i: (0, 0)),
        pl.BlockSpec((H, H), lambda i: (0, 0)),
        pl.BlockSpec((1, H), lambda i: (0, 0)),
        pl.BlockSpec((H, H), lambda i: (0, 0)),
        pl.BlockSpec((1, H), lambda i: (0, 0)),
    ],
    out_specs=pl.BlockSpec((_BLK, H), lambda i: (i, 0)),
    out_shape=jax.ShapeDtypeStruct((B, H), jnp.float32),
)


def kernel(inputs_BL, emb, W1, b1, W2, b2, W3, b3):
    tab = _pack(emb.T, emb.T, emb.T, emb.T,
                jnp.eye(E, dtype=jnp.float32))
    idx = inputs_BL.astype(jnp.int32).T.reshape(-1)
    q = idx // _SQ
    k = idx - q * _SQ
    rows = _sc_gather(k.reshape(_NW, _NCH, _CH), tab)
    qcol = q.reshape(_N, 1)
    w3p = jnp.pad(W3, ((0, 0), (0, H - C)))
    bf = jnp.bfloat16
    w1r = jnp.concatenate(
        [jnp.tile(W1[64 * l + 32 * hh:64 * l + 32 * (hh + 1), :], (4, 1))
         for l in range(L) for hh in (0, 1)], axis=0)
    out = _mlp(
        rows, rows, rows, rows, rows,
        qcol, qcol, qcol, qcol, qcol,
        w1r.astype(bf), b1.reshape(1, H),
        W2.astype(bf), b2.reshape(1, H),
        w3p.astype(bf), jnp.pad(b3, (0, H - C)).reshape(1, H),
    )
    return out[:, :C]


# speedup vs baseline: 1.4098x; 1.4098x over previous
import functools

import jax
import jax.numpy as jnp
from jax import lax
from jax.experimental import pallas as pl
from jax.experimental.pallas import tpu as pltpu
from jax.experimental.pallas import tpu_sc as plsc

B, L, V, E, H, C = 16384, 5, 1000000, 64, 128, 2


_VBQ = 8192
_SQ = 253952
_TGRID = _SQ // _VBQ
_MAXBLK = (V // _VBQ) - 1


def _pack_body(x0_ref, x1_ref, x2_ref, x3_ref, eye_ref, o_ref):
    quarters = []
    for x_ref in (x0_ref, x1_ref, x2_ref, x3_ref):
        xb = lax.bitcast_convert_type(x_ref[...], jnp.int32)
        lo = jnp.bitwise_and(lax.shift_right_logical(xb[:E // 2, :], 16),
                             jnp.int32(0xFFFF))
        hi = jnp.bitwise_and(xb[E // 2:, :], jnp.int32(-65536))
        quarters.append(jnp.bitwise_or(lo, hi))
    p = jnp.concatenate(quarters, axis=0)
    pf = lax.bitcast_convert_type(p, jnp.float32)
    t = lax.dot_general(pf, eye_ref[...], (((0,), (0,)), ((), ())),
                        preferred_element_type=jnp.float32)
    o_ref[...] = lax.bitcast_convert_type(t, jnp.int32)


def _in_spec(m):
    return pl.BlockSpec(
        (E, _VBQ),
        lambda i, _m=m: (0, jnp.minimum(_m * _TGRID + i, _MAXBLK)))


_pack = pl.pallas_call(
    _pack_body,
    grid=(_TGRID,),
    in_specs=[_in_spec(m) for m in range(4)] + [
        pl.BlockSpec((2 * E, 2 * E), lambda i: (0, 0)),
    ],
    out_specs=pl.BlockSpec((_VBQ, 2 * E), lambda i: (i, 0)),
    out_shape=jax.ShapeDtypeStruct((_SQ, 2 * E), jnp.int32),
)


_NC, _NS = 2, 16
_NW = _NC * _NS
_N = B * L
_PW = _N // _NW
_CH = 128
_NCH = _PW // _CH
_NBUF = 6

_sc_mesh = plsc.VectorSubcoreMesh(core_axis_name="c", subcore_axis_name="s")


@functools.partial(
    pl.kernel,
    out_type=jax.ShapeDtypeStruct((_N, 2 * E), jnp.int32),
    mesh=_sc_mesh,
    scratch_types=[
        pltpu.VMEM((_NCH, _CH), jnp.int32),
        pltpu.VMEM((_NBUF, _CH, 2 * E), jnp.int32),
    ] + [pltpu.SemaphoreType.DMA] * _NBUF,
)
def _sc_gather(idx_hbm, tab_hbm, out_hbm, idx_v, rows_v, *sems):
    wid = lax.axis_index("s") * _NC + lax.axis_index("c")
    base = wid * _PW
    pltpu.sync_copy(idx_hbm.at[wid], idx_v)

    def start(i):
        bi = i % _NBUF
        return pltpu.async_copy(tab_hbm.at[idx_v.at[i]], rows_v.at[bi], sems[bi])

    handles = {}
    for i in range(min(_NBUF, _NCH)):
        handles[i] = start(i)
    for i in range(_NCH):
        bi = i % _NBUF
        handles.pop(i).wait()
        pltpu.sync_copy(rows_v.at[bi], out_hbm.at[pl.ds(base + i * _CH, _CH)])
        j = i + _NBUF
        if j < _NCH:
            handles[j] = start(j)


_BLK = 2048
_NB = B // _BLK


def _mlp_t_body(x_refs, q_refs, eye_ref, w1t_ref, b1_ref, w2t_ref, b2_ref,
                w3t_ref, b3_ref, o_ref):
    eye = eye_ref[...]
    planes = []
    for l in range(L):
        xf = lax.bitcast_convert_type(x_refs[l][...], jnp.float32)
        xt = lax.dot_general(eye, xf, (((1,), (1,)), ((), ())),
                             preferred_element_type=jnp.float32)
        xi = lax.bitcast_convert_type(xt, jnp.int32)
        qv = q_refs[l][...]
        sel = jnp.zeros((E // 2, _BLK), jnp.int32)
        for q in range(4):
            sel = jnp.where(qv == q, xi[32 * q:32 * (q + 1), :], sel)
        f_lo = lax.bitcast_convert_type(
            lax.shift_left(sel, 16), jnp.float32)
        f_hi = lax.bitcast_convert_type(
            jnp.bitwise_and(sel, jnp.int32(-65536)), jnp.float32)
        planes += [f_lo, f_hi]
    x = jnp.concatenate(planes, axis=0).astype(jnp.bfloat16)
    acc = jnp.dot(w1t_ref[...], x, preferred_element_type=jnp.float32)
    h = jnp.maximum(acc + b1_ref[...], 0.0).astype(jnp.bfloat16)
    h = jnp.maximum(jnp.dot(w2t_ref[...], h,
                            preferred_element_type=jnp.float32) + b2_ref[...], 0.0)
    o = jnp.dot(w3t_ref[...], h.astype(jnp.bfloat16),
                preferred_element_type=jnp.float32) + b3_ref[...]
    row = lax.broadcasted_iota(jnp.int32, o.shape, 0)
    o = jnp.where(row < C, o, jnp.float32(-1e30))
    m = jnp.max(o, axis=0, keepdims=True)
    e = jnp.exp(o - m)
    o_ref[...] = e / jnp.sum(e, axis=0, keepdims=True)


def _mlp_t_entry(x0, x1, x2, x3, x4, q0, q1, q2, q3, q4,
                 eye, w1t, b1, w2t, b2, w3t, b3, o):
    _mlp_t_body((x0, x1, x2, x3, x4), (q0, q1, q2, q3, q4),
                eye, w1t, b1, w2t, b2, w3t, b3, o)


def _x_spec(l):
    return pl.BlockSpec((_BLK, 2 * E), lambda i, _l=l: (_l * _NB + i, 0))


def _q_spec(l):
    return pl.BlockSpec((1, _BLK), lambda i, _l=l: (0, _l * _NB + i))


_mlp_t = pl.pallas_call(
    _mlp_t_entry,
    grid=(_NB,),
    in_specs=[_x_spec(l) for l in range(L)]
    + [_q_spec(l) for l in range(L)] + [
        pl.BlockSpec((2 * E, 2 * E), lambda i: (0, 0)),
        pl.BlockSpec((H, L * E), lambda i: (0, 0)),
        pl.BlockSpec((H, 1), lambda i: (0, 0)),
        pl.BlockSpec((H, H), lambda i: (0, 0)),
        pl.BlockSpec((H, 1), lambda i: (0, 0)),
        pl.BlockSpec((H, H), lambda i: (0, 0)),
        pl.BlockSpec((H, 1), lambda i: (0, 0)),
    ],
    out_specs=pl.BlockSpec((H, _BLK), lambda i: (0, i)),
    out_shape=jax.ShapeDtypeStruct((H, B), jnp.float32),
)


def kernel(inputs_BL, emb, W1, b1, W2, b2, W3, b3):
    tab = _pack(emb.T, emb.T, emb.T, emb.T,
                jnp.eye(E, dtype=jnp.float32))
    idx = inputs_BL.astype(jnp.int32).T.reshape(-1)
    q = idx // _SQ
    k = idx - q * _SQ
    rows = _sc_gather(k.reshape(_NW, _NCH, _CH), tab)
    qrow = q.reshape(1, _N)
    w3p = jnp.pad(W3, ((0, 0), (0, H - C)))
    bf = jnp.bfloat16
    oT = _mlp_t(
        rows, rows, rows, rows, rows,
        qrow, qrow, qrow, qrow, qrow,
        jnp.eye(2 * E, dtype=jnp.float32),
        W1.T.astype(bf), b1.reshape(H, 1),
        W2.T.astype(bf), b2.reshape(H, 1),
        w3p.T.astype(bf), jnp.pad(b3, (0, H - C)).reshape(H, 1),
    )
    return oT[:C, :].T
